# baseline (speedup 1.0000x reference)
import functools
import math

import jax
import jax.numpy as jnp
from jax import lax
from jax.experimental import pallas as pl
from jax.experimental.pallas import tpu as pltpu
from jax.experimental.pallas import tpu_sc as plsc

HIDDEN = 128
EPS = 1e-3

_GATHER_WINDOW = 128


def _sc_gather(table, idx_flat):
    n = idx_flat.shape[0]
    idx2 = idx_flat.reshape(1, n)
    mesh = plsc.VectorSubcoreMesh(core_axis_name="core", subcore_axis_name="subcore")

    @functools.partial(
        pl.kernel,
        out_type=jax.ShapeDtypeStruct((n, HIDDEN), table.dtype),
        mesh=mesh,
    )
    def gather_kernel(table_hbm, idx_hbm, out_hbm):
        def body(idx_vmem, out_vmem):
            pltpu.sync_copy(table_hbm.at[idx_vmem.at[0]], out_vmem)

        pltpu.emit_pipeline(
            body,
            grid=(n // _GATHER_WINDOW,),
            in_specs=[
                pl.BlockSpec((1, _GATHER_WINDOW), index_map=lambda i: (0, i))
            ],
            out_specs=[
                pl.BlockSpec((_GATHER_WINDOW, HIDDEN), index_map=lambda i: (i, 0))
            ],
            core_axis_name=("core", "subcore"),
            dimension_semantics=(pltpu.PARALLEL,),
        )(idx_hbm, out_hbm)

    return gather_kernel(table, idx2)


def _ln_body(g_ref, seg_ref, pos_ref, segtab_ref, gamma_ref, beta_ref, out_ref):
    bb, s, hdim = g_ref.shape
    g = g_ref[...]
    seg = seg_ref[...]
    seg0 = segtab_ref[0, :]
    seg1 = segtab_ref[1, :]
    pos2 = pos_ref[...] + seg0[None, :]
    segf = seg.astype(jnp.float32)[..., None]
    h = g * math.sqrt(float(HIDDEN)) + pos2[None, :, :] + segf * (seg1 - seg0)
    hf = h.reshape(bb * s, hdim)
    ones = jnp.ones((hdim, 128), dtype=jnp.float32)
    s1 = lax.dot_general(hf, ones, (((1,), (0,)), ((), ())),
                         preferred_element_type=jnp.float32)[:, :1]
    s2 = lax.dot_general(hf * hf, ones, (((1,), (0,)), ((), ())),
                         preferred_element_type=jnp.float32)[:, :1]
    mean = s1 * (1.0 / hdim)
    var = s2 * (1.0 / hdim) - mean * mean
    r = lax.rsqrt(var + EPS)
    d = mean * r
    out = (hf * r - d) * gamma_ref[...][None, :] + beta_ref[...][None, :]
    out_ref[...] = out.reshape(bb, s, hdim)


def kernel(x, seg, emb_table, pos_table, seg_table, gamma, beta):
    b, s = x.shape
    n = b * s
    gathered = _sc_gather(emb_table, x.reshape(n).astype(jnp.int32))
    gathered = gathered.reshape(b, s, HIDDEN)
    pos = pos_table[:s]

    bb = 16
    grid = (b // bb,)
    out = pl.pallas_call(
        _ln_body,
        grid=grid,
        in_specs=[
            pl.BlockSpec((bb, s, HIDDEN), lambda i: (i, 0, 0)),
            pl.BlockSpec((bb, s), lambda i: (i, 0)),
            pl.BlockSpec((s, HIDDEN), lambda i: (0, 0)),
            pl.BlockSpec((2, HIDDEN), lambda i: (0, 0)),
            pl.BlockSpec((HIDDEN,), lambda i: (0,)),
            pl.BlockSpec((HIDDEN,), lambda i: (0,)),
        ],
        out_specs=pl.BlockSpec((bb, s, HIDDEN), lambda i: (i, 0, 0)),
        out_shape=jax.ShapeDtypeStruct((b, s, HIDDEN), jnp.float32),
    )(gathered, seg.astype(jnp.int32), pos, seg_table, gamma, beta)
    return out


# speedup vs baseline: 1.2228x; 1.2228x over previous
import functools
import math

import jax
import jax.numpy as jnp
from jax import lax
from jax.experimental import pallas as pl
from jax.experimental.pallas import tpu as pltpu
from jax.experimental.pallas import tpu_sc as plsc

HIDDEN = 128
EPS = 1e-3

_GATHER_WINDOW = 128


def _sc_gather(table, idx_flat):
    n = idx_flat.shape[0]
    idx2 = idx_flat.reshape(1, n)
    mesh = plsc.VectorSubcoreMesh(core_axis_name="core", subcore_axis_name="subcore")

    @functools.partial(
        pl.kernel,
        out_type=jax.ShapeDtypeStruct((n, HIDDEN), table.dtype),
        mesh=mesh,
    )
    def gather_kernel(table_hbm, idx_hbm, out_hbm):
        def body(idx_vmem, out_vmem):
            pltpu.sync_copy(table_hbm.at[idx_vmem.at[0]], out_vmem)

        pltpu.emit_pipeline(
            body,
            grid=(n // _GATHER_WINDOW,),
            in_specs=[
                pl.BlockSpec((1, _GATHER_WINDOW), index_map=lambda i: (0, i))
            ],
            out_specs=[
                pl.BlockSpec((_GATHER_WINDOW, HIDDEN), index_map=lambda i: (i, 0))
            ],
            core_axis_name=("core", "subcore"),
            dimension_semantics=(pltpu.PARALLEL,),
        )(idx_hbm, out_hbm)

    return gather_kernel(table, idx2)


def _ln_body(g_ref, seg_ref, pos_ref, segtab_ref, gamma_ref, beta_ref, out_ref):
    g = g_ref[...]
    seg = seg_ref[...]
    pos = pos_ref[...]
    seg0 = segtab_ref[0, :]
    seg1 = segtab_ref[1, :]
    h = g * math.sqrt(float(HIDDEN)) + pos[None, :, :]
    h = h + jnp.where((seg[..., None] == 0), seg0, seg1)
    mean = jnp.mean(h, axis=-1, keepdims=True)
    var = jnp.mean((h - mean) * (h - mean), axis=-1, keepdims=True)
    out = (h - mean) * lax.rsqrt(var + EPS)
    out_ref[...] = out * gamma_ref[...] + beta_ref[...]


def _aliased_ln_body(buf_ref, g_ref, seg_ref, pos_ref, segtab_ref, gamma_ref,
                     beta_ref, out_ref):
    del buf_ref
    _ln_body(g_ref, seg_ref, pos_ref, segtab_ref, gamma_ref, beta_ref, out_ref)


_N_CHUNKS = 4
_BB = 16


def _ln_chunk(buf, chunk_idx, b_full, g, seg, pos, seg_table, gamma, beta):
    s = g.shape[1]
    bc = g.shape[0]
    blk0 = chunk_idx * (bc // _BB)
    data_specs = [
        pl.BlockSpec((_BB, s, HIDDEN), lambda i: (i, 0, 0)),
        pl.BlockSpec((_BB, s), lambda i: (i, 0)),
        pl.BlockSpec((s, HIDDEN), lambda i: (0, 0)),
        pl.BlockSpec((2, HIDDEN), lambda i: (0, 0)),
        pl.BlockSpec((HIDDEN,), lambda i: (0,)),
        pl.BlockSpec((HIDDEN,), lambda i: (0,)),
    ]
    common = dict(
        grid=(bc // _BB,),
        out_specs=pl.BlockSpec((_BB, s, HIDDEN), lambda i: (i + blk0, 0, 0)),
        out_shape=jax.ShapeDtypeStruct((b_full, s, HIDDEN), jnp.float32),
    )
    if buf is None:
        return pl.pallas_call(_ln_body, in_specs=data_specs, **common)(
            g, seg, pos, seg_table, gamma, beta)
    return pl.pallas_call(
        _aliased_ln_body,
        in_specs=[pl.BlockSpec(memory_space=pl.ANY)] + data_specs,
        input_output_aliases={0: 0},
        **common,
    )(buf, g, seg, pos, seg_table, gamma, beta)


def kernel(x, seg, emb_table, pos_table, seg_table, gamma, beta):
    b, s = x.shape
    bc = b // _N_CHUNKS
    xi = x.astype(jnp.int32)
    segi = seg.astype(jnp.int32)
    pos = pos_table[:s]

    gs = [
        _sc_gather(emb_table, xi[c * bc:(c + 1) * bc].reshape(bc * s))
        .reshape(bc, s, HIDDEN)
        for c in range(_N_CHUNKS)
    ]

    buf = None
    for c in range(_N_CHUNKS):
        buf = _ln_chunk(buf, c, b, gs[c], segi[c * bc:(c + 1) * bc], pos,
                        seg_table, gamma, beta)
    return buf
